# baseline (speedup 1.0000x reference)
import jax
import jax.numpy as jnp
from jax import lax
from jax.experimental import pallas as pl
from jax.experimental.pallas import tpu as pltpu
from jax.experimental.pallas import tpu_sc as plsc

N = 10000
E = 160000
D = 256
DH = D // 2
NT = 16
EP = 163840
EPT = EP // NT
CH = 32
NCHUNK = EPT // CH
NB = 4
MB = 2 * NB
T = NCHUNK // NB
RPT = 640
RPT_LAST = N - (NT - 1) * RPT
BR = 32

BM = 1000


def _mm_body(h_ref, w_ref, n_ref, o_ref):
    hw = jnp.dot(h_ref[...].astype(jnp.bfloat16),
                 w_ref[...].astype(jnp.bfloat16),
                 preferred_element_type=jnp.float32)
    o_ref[...] = hw * n_ref[...]


def _tc_hs2(h, weight, norm):
    grid = (N // BM, 2)
    return pl.pallas_call(
        _mm_body,
        grid=grid,
        in_specs=[
            pl.BlockSpec((BM, D), lambda i, j: (i, 0)),
            pl.BlockSpec((D, DH), lambda i, j: (0, j)),
            pl.BlockSpec((BM, 1), lambda i, j: (i, 0)),
        ],
        out_specs=pl.BlockSpec((BM, DH), lambda i, j: (j * (N // BM) + i, 0)),
        out_shape=jax.ShapeDtypeStruct((2 * N, DH), jnp.float32),
    )(h, weight, norm)


def _sc_body(hs2, esrc2, edst, ew, norm1, bias1, out,
             dstring, srcring, wring, rows, sbuf, nbuf, bbuf, agg,
             gsem, ssem, msem, dsem):
    c = lax.axis_index("c")
    s = lax.axis_index("s")

    zv = jnp.zeros((16,), jnp.float32)
    for r in range(BR):
        for d in range(DH // 16):
            rows[0][r, pl.ds(d * 16, 16)] = zv
    row0 = s * RPT

    def zblk(k, carry):
        pltpu.async_copy(rows[0], agg.at[pl.ds(row0 + k * BR, BR)], gsem[0])
        return carry

    def zwait(k, carry):
        pltpu.make_async_copy(rows[0], agg.at[pl.ds(row0, BR)],
                              gsem[0]).wait()
        return carry

    nz = lax.select(s == NT - 1, RPT_LAST // BR, RPT // BR)
    lax.fori_loop(0, nz, zblk, 0)
    lax.fori_loop(0, nz, zwait, 0)
    @pl.when(s == NT - 1)
    def _():
        pltpu.sync_copy(rows[0].at[pl.ds(0, RPT_LAST % BR)],
                        agg.at[pl.ds(row0 + (RPT_LAST // BR) * BR,
                                     RPT_LAST % BR)])

    @pl.when(s < NT - 1)
    def _():
        pltpu.sync_copy(norm1.at[pl.ds(row0, RPT)], nbuf)

    @pl.when(s == NT - 1)
    def _():
        pltpu.sync_copy(norm1.at[pl.ds(row0, RPT_LAST)],
                        nbuf.at[pl.ds(0, RPT_LAST)])

    pltpu.sync_copy(bias1.at[pl.ds(c * DH, DH)], bbuf)

    coff = c * N
    wbase = s * EPT

    def issue_dst(g, m):
        pltpu.async_copy(edst.at[pl.ds(wbase + g * CH, CH)],
                         dstring[m], dsem[m])

    def wait_dst(m):
        pltpu.make_async_copy(edst.at[pl.ds(0, CH)],
                              dstring[m], dsem[m]).wait()

    def issue_meta(g, m):
        pltpu.async_copy(esrc2.at[pl.ds(wbase + g * CH, CH)],
                         srcring[m], msem[m])
        pltpu.async_copy(ew.at[pl.ds(wbase + g * CH, CH)],
                         wring[m], msem[m])

    def wait_meta(m):
        pltpu.make_async_copy(esrc2.at[pl.ds(0, CH)],
                              srcring[m], msem[m]).wait()
        pltpu.make_async_copy(ew.at[pl.ds(0, CH)], wring[m], msem[m]).wait()
        for q in range(CH // 16):
            sl = pl.ds(q * 16, 16)
            srcring[m][sl] = srcring[m][sl] + coff

    def issue_gather(m, b):
        pltpu.async_copy(hs2.at[srcring[m]], rows[b], gsem[b])

    def wait_gather(b):
        pltpu.make_async_copy(hs2.at[srcring[0]], rows[b], gsem[b]).wait()

    def issue_scatter(m, b):
        pltpu.async_copy(sbuf[b], agg.at[dstring[m]], ssem[b], add=True)

    def wait_scatter(b):
        pltpu.make_async_copy(sbuf[b], agg.at[dstring[0]], ssem[b]).wait()

    def scale(m, b):
        def grp(j, cc):
            wv = wring[m][pl.ds(j * 16, 16)]
            for l in range(16):
                w = wv[l]
                e = j * 16 + l
                for d in range(DH // 16):
                    sl = pl.ds(d * 16, 16)
                    sbuf[b][e, sl] = rows[b][e, sl] * w
            return cc

        lax.fori_loop(0, CH // 16, grp, 0)

    plsc.subcore_barrier()

    for m in range(MB):
        issue_meta(m, m)
    for b in range(NB):
        issue_dst(b, b)
        wait_meta(b)
        issue_gather(b, b)

    def outer(t, carry):
        for p in range(2):
            for b in range(NB):
                g = (2 * t + p) * NB + b
                m = (p * NB + b) % MB
                m_next = (m + NB) % MB
                wait_gather(b)

                @pl.when(g >= NB)
                def _():
                    wait_scatter(b)

                scale(m, b)

                @pl.when(g + NB < NCHUNK)
                def _():
                    issue_dst(g + NB, m_next)

                wait_dst(m)
                issue_scatter(m, b)

                @pl.when(g + MB < NCHUNK)
                def _():
                    issue_meta(g + MB, m)

                @pl.when(g + NB < NCHUNK)
                def _():
                    wait_meta(m_next)
                    issue_gather(m_next, b)

        return carry

    lax.fori_loop(0, T // 2, outer, 0)
    for b in range(NB):
        wait_scatter(b)
    plsc.subcore_barrier()

    bb = [bbuf[pl.ds(d * 16, 16)] for d in range(DH // 16)]

    def wcompute(buf, k):
        for j in range(BR // 16):
            nv = nbuf[pl.ds(k * BR + j * 16, 16)]
            for l in range(16):
                w = nv[l]
                r = j * 16 + l
                for d in range(DH // 16):
                    sl = pl.ds(d * 16, 16)
                    buf[r, sl] = buf[r, sl] * w + bb[d]

    def wout(t, carry):
        for p in range(2):
            k = 2 * t + p

            @pl.when(t > 0)
            def _():
                pltpu.make_async_copy(
                    sbuf[p], out.at[pl.ds(row0, BR), pl.ds(c * DH, DH)],
                    ssem[p]).wait()

            pltpu.sync_copy(agg.at[pl.ds(row0 + k * BR, BR)], sbuf[p])
            wcompute(sbuf[p], k)
            pltpu.async_copy(
                sbuf[p],
                out.at[pl.ds(row0 + k * BR, BR), pl.ds(c * DH, DH)],
                ssem[p])
        return carry

    nw = lax.select(s == NT - 1, RPT_LAST // BR, RPT // BR)
    lax.fori_loop(0, nw // 2, wout, 0)
    for p in range(2):
        pltpu.make_async_copy(
            sbuf[p], out.at[pl.ds(row0, BR), pl.ds(c * DH, DH)],
            ssem[p]).wait()

    @pl.when(s == NT - 1)
    def _():
        kt = RPT_LAST // BR
        pltpu.sync_copy(agg.at[pl.ds(row0 + kt * BR, 16)],
                        sbuf[0].at[pl.ds(0, 16)])
        nv = nbuf[pl.ds(kt * BR, 16)]
        for l in range(16):
            w = nv[l]
            for d in range(DH // 16):
                sl = pl.ds(d * 16, 16)
                sbuf[0][l, sl] = sbuf[0][l, sl] * w + bb[d]
        pltpu.sync_copy(sbuf[0].at[pl.ds(0, 16)],
                        out.at[pl.ds(row0 + kt * BR, 16),
                               pl.ds(c * DH, DH)])


_sc_agg = pl.kernel(
    _sc_body,
    out_type=jax.ShapeDtypeStruct((N, D), jnp.float32),
    mesh=plsc.VectorSubcoreMesh(core_axis_name="c", subcore_axis_name="s"),
    scratch_types=[
        [pltpu.VMEM((CH,), jnp.int32) for _ in range(MB)],
        [pltpu.VMEM((CH,), jnp.int32) for _ in range(MB)],
        [pltpu.VMEM((CH,), jnp.float32) for _ in range(MB)],
        [pltpu.VMEM((CH, DH), jnp.float32) for _ in range(NB)],
        [pltpu.VMEM((CH, DH), jnp.float32) for _ in range(NB)],
        pltpu.VMEM((RPT,), jnp.float32),
        pltpu.VMEM((DH,), jnp.float32),
        pltpu.VMEM_SHARED((N, DH), jnp.float32),
        [pltpu.SemaphoreType.DMA for _ in range(NB)],
        [pltpu.SemaphoreType.DMA for _ in range(NB)],
        [pltpu.SemaphoreType.DMA for _ in range(MB)],
        [pltpu.SemaphoreType.DMA for _ in range(MB)],
    ],
)


def kernel(h, edge_index, norm, edge_weight, weight, bias):
    hs2 = _tc_hs2(h, weight, norm)
    pad = EP - E
    srcp = jnp.concatenate([edge_index[0], jnp.zeros((pad,), jnp.int32)])
    edstp = jnp.concatenate([edge_index[1], jnp.zeros((pad,), jnp.int32)])
    ewp = jnp.concatenate([edge_weight, jnp.zeros((pad,), jnp.float32)])
    return _sc_agg(hs2, srcp, edstp, ewp, norm.reshape(N), bias)


# speedup vs baseline: 1.0851x; 1.0851x over previous
import jax
import jax.numpy as jnp
from jax import lax
from jax.experimental import pallas as pl
from jax.experimental.pallas import tpu as pltpu
from jax.experimental.pallas import tpu_sc as plsc

N = 10000
E = 160000
D = 256
DH = D // 2
NT = 16
EP = 163840
EPT = EP // NT
CH = 32
NCHUNK = EPT // CH
NB = 4
MB = 2 * NB
T = NCHUNK // NB
RPT = 640
RPT_LAST = N - (NT - 1) * RPT
BR = 32

BM = 1000


def _mm_body(h_ref, w_ref, n_ref, o_ref):
    hw = jnp.dot(h_ref[...].astype(jnp.bfloat16),
                 w_ref[...].astype(jnp.bfloat16),
                 preferred_element_type=jnp.float32)
    o_ref[...] = hw * n_ref[...]


def _tc_hs2(h, weight, norm):
    grid = (N // BM, 2)
    return pl.pallas_call(
        _mm_body,
        grid=grid,
        in_specs=[
            pl.BlockSpec((BM, D), lambda i, j: (i, 0)),
            pl.BlockSpec((D, DH), lambda i, j: (0, j)),
            pl.BlockSpec((BM, 1), lambda i, j: (i, 0)),
        ],
        out_specs=pl.BlockSpec((BM, DH), lambda i, j: (j * (N // BM) + i, 0)),
        out_shape=jax.ShapeDtypeStruct((2 * N, DH), jnp.float32),
    )(h, weight, norm)


def _sc_body(hs2, esrc2, edst, ew, norm1, bias1, out,
             dstring, srcring, wring, rows, sbuf, nbuf, bbuf, agg,
             gsem, ssem, msem, dsem):
    c = lax.axis_index("c")
    s = lax.axis_index("s")

    zv = jnp.zeros((16,), jnp.float32)
    for r in range(BR):
        for d in range(DH // 16):
            rows[0][r, pl.ds(d * 16, 16)] = zv
    row0 = s * RPT

    def zblk(k, carry):
        pltpu.async_copy(rows[0], agg.at[pl.ds(row0 + k * BR, BR)], gsem[0])
        return carry

    def zwait(k, carry):
        pltpu.make_async_copy(rows[0], agg.at[pl.ds(row0, BR)],
                              gsem[0]).wait()
        return carry

    nz = lax.select(s == NT - 1, RPT_LAST // BR, RPT // BR)
    lax.fori_loop(0, nz, zblk, 0)
    lax.fori_loop(0, nz, zwait, 0)
    @pl.when(s == NT - 1)
    def _():
        pltpu.sync_copy(rows[0].at[pl.ds(0, RPT_LAST % BR)],
                        agg.at[pl.ds(row0 + (RPT_LAST // BR) * BR,
                                     RPT_LAST % BR)])

    @pl.when(s < NT - 1)
    def _():
        pltpu.sync_copy(norm1.at[pl.ds(row0, RPT)], nbuf)

    @pl.when(s == NT - 1)
    def _():
        pltpu.sync_copy(norm1.at[pl.ds(row0, RPT_LAST)],
                        nbuf.at[pl.ds(0, RPT_LAST)])

    pltpu.sync_copy(bias1.at[pl.ds(c * DH, DH)], bbuf)

    mbase = c * EP + s * EPT
    wbase = s * EPT

    def issue_dst(g, m):
        pltpu.async_copy(edst.at[pl.ds(wbase + g * CH, CH)],
                         dstring[m], dsem[m])

    def wait_dst(m):
        pltpu.make_async_copy(edst.at[pl.ds(0, CH)],
                              dstring[m], dsem[m]).wait()

    def issue_meta(g, m):
        pltpu.async_copy(esrc2.at[pl.ds(mbase + g * CH, CH)],
                         srcring[m], msem[m])
        pltpu.async_copy(ew.at[pl.ds(wbase + g * CH, CH)],
                         wring[m], msem[m])

    def wait_meta(m):
        pltpu.make_async_copy(esrc2.at[pl.ds(0, CH)],
                              srcring[m], msem[m]).wait()
        pltpu.make_async_copy(ew.at[pl.ds(0, CH)], wring[m], msem[m]).wait()

    def issue_gather(m, b):
        pltpu.async_copy(hs2.at[srcring[m]], rows[b], gsem[b])

    def wait_gather(b):
        pltpu.make_async_copy(hs2.at[srcring[0]], rows[b], gsem[b]).wait()

    def issue_scatter(m, b):
        pltpu.async_copy(sbuf[b], agg.at[dstring[m]], ssem[b], add=True)

    def wait_scatter(b):
        pltpu.make_async_copy(sbuf[b], agg.at[dstring[0]], ssem[b]).wait()

    def scale(m, b):
        def grp(j, cc):
            wv = wring[m][pl.ds(j * 16, 16)]
            for l in range(16):
                w = wv[l]
                e = j * 16 + l
                for d in range(DH // 16):
                    sl = pl.ds(d * 16, 16)
                    sbuf[b][e, sl] = rows[b][e, sl] * w
            return cc

        lax.fori_loop(0, CH // 16, grp, 0)

    plsc.subcore_barrier()

    for m in range(MB):
        issue_meta(m, m)
    for b in range(NB):
        issue_dst(b, b)
        wait_meta(b)
        issue_gather(b, b)

    def outer(t, carry):
        for p in range(2):
            for b in range(NB):
                g = (2 * t + p) * NB + b
                m = (p * NB + b) % MB
                m_next = (m + NB) % MB
                wait_gather(b)

                @pl.when(g >= NB)
                def _():
                    wait_scatter(b)

                scale(m, b)

                @pl.when(g + NB < NCHUNK)
                def _():
                    issue_dst(g + NB, m_next)

                wait_dst(m)
                issue_scatter(m, b)

                @pl.when(g + MB < NCHUNK)
                def _():
                    issue_meta(g + MB, m)

                @pl.when(g + NB < NCHUNK)
                def _():
                    wait_meta(m_next)
                    issue_gather(m_next, b)

        return carry

    lax.fori_loop(0, T // 2, outer, 0)
    for b in range(NB):
        wait_scatter(b)
    plsc.subcore_barrier()

    bb = [bbuf[pl.ds(d * 16, 16)] for d in range(DH // 16)]

    def wcompute(buf, k):
        for j in range(BR // 16):
            nv = nbuf[pl.ds(k * BR + j * 16, 16)]
            for l in range(16):
                w = nv[l]
                r = j * 16 + l
                for d in range(DH // 16):
                    sl = pl.ds(d * 16, 16)
                    buf[r, sl] = buf[r, sl] * w + bb[d]

    def wout(t, carry):
        for p in range(2):
            k = 2 * t + p

            @pl.when(t > 0)
            def _():
                pltpu.make_async_copy(
                    sbuf[p], out.at[pl.ds(row0, BR), pl.ds(c * DH, DH)],
                    ssem[p]).wait()

            pltpu.sync_copy(agg.at[pl.ds(row0 + k * BR, BR)], sbuf[p])
            wcompute(sbuf[p], k)
            pltpu.async_copy(
                sbuf[p],
                out.at[pl.ds(row0 + k * BR, BR), pl.ds(c * DH, DH)],
                ssem[p])
        return carry

    nw = lax.select(s == NT - 1, RPT_LAST // BR, RPT // BR)
    lax.fori_loop(0, nw // 2, wout, 0)
    for p in range(2):
        pltpu.make_async_copy(
            sbuf[p], out.at[pl.ds(row0, BR), pl.ds(c * DH, DH)],
            ssem[p]).wait()

    @pl.when(s == NT - 1)
    def _():
        kt = RPT_LAST // BR
        pltpu.sync_copy(agg.at[pl.ds(row0 + kt * BR, 16)],
                        sbuf[0].at[pl.ds(0, 16)])
        nv = nbuf[pl.ds(kt * BR, 16)]
        for l in range(16):
            w = nv[l]
            for d in range(DH // 16):
                sl = pl.ds(d * 16, 16)
                sbuf[0][l, sl] = sbuf[0][l, sl] * w + bb[d]
        pltpu.sync_copy(sbuf[0].at[pl.ds(0, 16)],
                        out.at[pl.ds(row0 + kt * BR, 16),
                               pl.ds(c * DH, DH)])


_sc_agg = pl.kernel(
    _sc_body,
    out_type=jax.ShapeDtypeStruct((N, D), jnp.float32),
    mesh=plsc.VectorSubcoreMesh(core_axis_name="c", subcore_axis_name="s"),
    scratch_types=[
        [pltpu.VMEM((CH,), jnp.int32) for _ in range(MB)],
        [pltpu.VMEM((CH,), jnp.int32) for _ in range(MB)],
        [pltpu.VMEM((CH,), jnp.float32) for _ in range(MB)],
        [pltpu.VMEM((CH, DH), jnp.float32) for _ in range(NB)],
        [pltpu.VMEM((CH, DH), jnp.float32) for _ in range(NB)],
        pltpu.VMEM((RPT,), jnp.float32),
        pltpu.VMEM((DH,), jnp.float32),
        pltpu.VMEM_SHARED((N, DH), jnp.float32),
        [pltpu.SemaphoreType.DMA for _ in range(NB)],
        [pltpu.SemaphoreType.DMA for _ in range(NB)],
        [pltpu.SemaphoreType.DMA for _ in range(MB)],
        [pltpu.SemaphoreType.DMA for _ in range(MB)],
    ],
)


def kernel(h, edge_index, norm, edge_weight, weight, bias):
    hs2 = _tc_hs2(h, weight, norm)
    pad = EP - E
    srcp = jnp.concatenate([edge_index[0], jnp.zeros((pad,), jnp.int32)])
    esrc2 = jnp.concatenate([srcp, srcp + N])
    edstp = jnp.concatenate([edge_index[1], jnp.zeros((pad,), jnp.int32)])
    ewp = jnp.concatenate([edge_weight, jnp.zeros((pad,), jnp.float32)])
    return _sc_agg(hs2, esrc2, edstp, ewp, norm.reshape(N), bias)
